# baseline (speedup 1.0000x reference)
import jax
import jax.numpy as jnp
from jax.experimental import pallas as pl
from jax.experimental.pallas import tpu as pltpu

_B, _NIN, _NHID, _E = 2048, 1024, 1024, 8
_TILE = 128
_NT = 24
_C = _NT * _TILE


def _mm_body(te_ref, x_ref, w_ref, b_ref, o_ref):
    y = jax.lax.dot_general(
        x_ref[...], w_ref[0],
        dimension_numbers=(((1,), (1,)), ((), ())),
        preferred_element_type=jnp.float32)
    o_ref[...] = jnp.maximum(y + b_ref[0], 0.0)


def _expert_matmul(x_sorted, tile_expert, W, b):
    grid_spec = pltpu.PrefetchScalarGridSpec(
        num_scalar_prefetch=1,
        grid=(_NT,),
        in_specs=[
            pl.BlockSpec((_TILE, _NIN), lambda t, te: (t, 0)),
            pl.BlockSpec((1, _NHID, _NIN), lambda t, te: (te[t], 0, 0)),
            pl.BlockSpec((1, 1, _NHID), lambda t, te: (te[t], 0, 0)),
        ],
        out_specs=pl.BlockSpec((_TILE, _NHID), lambda t, te: (t, 0)),
    )
    return pl.pallas_call(
        _mm_body,
        grid_spec=grid_spec,
        out_shape=jax.ShapeDtypeStruct((_C, _NHID), jnp.float32),
    )(tile_expert, x_sorted, W, b.reshape(_E, 1, _NHID))


def kernel(x, slct, W, b):
    idx = jnp.argmax(slct, axis=1).astype(jnp.int32)
    oh = (idx[:, None] == jnp.arange(_E, dtype=jnp.int32)[None, :]).astype(jnp.int32)
    counts = jnp.sum(oh, axis=0)
    padded = ((counts + _TILE - 1) // _TILE) * _TILE
    ends = jnp.cumsum(padded)
    offs = ends - padded
    ranks = jnp.cumsum(oh, axis=0) - oh
    rank = jnp.sum(ranks * oh, axis=1)
    slot = offs[idx] + rank
    tok_map = jnp.zeros((_C,), jnp.int32).at[slot].set(
        jnp.arange(_B, dtype=jnp.int32))
    x_sorted = x[tok_map]
    tstart = jnp.arange(_NT, dtype=jnp.int32) * _TILE
    tile_expert = jnp.minimum(
        jnp.sum((tstart[:, None] >= ends[None, :]).astype(jnp.int32), axis=1),
        _E - 1).astype(jnp.int32)
    y_sorted = _expert_matmul(x_sorted, tile_expert, W, b)
    return y_sorted[slot]


# speedup vs baseline: 1.2516x; 1.2516x over previous
import jax
import jax.numpy as jnp
from jax import lax
from jax.experimental import pallas as pl
from jax.experimental.pallas import tpu as pltpu
from jax.experimental.pallas import tpu_sc as plsc

_B, _NIN, _NHID, _E = 2048, 1024, 1024, 8
_TILE = 128
_NT = 24
_C = _NT * _TILE

_NC, _NS = 2, 16
_NW = _NC * _NS
_TPW = _B // _NW
_G = _TPW // 16

_MESH = plsc.VectorSubcoreMesh(core_axis_name="c", subcore_axis_name="s",
                               num_cores=_NC, num_subcores=_NS)


def _i16(v):
    return jnp.full((16,), v, jnp.int32)


def _route_body(slct_t, tm_hbm, meta_hbm,
                sbuf, fbuf, idxb, rankb, slotb, cntv, cntf, cntall, svm, fillm,
                endm, teb, tmbuf, csh, sem, fsem):
    cid = lax.axis_index("c")
    sid = lax.axis_index("s")
    wid = cid * _NS + sid
    fwid = (1 - cid) * _NS + sid
    base = wid * _TPW
    fbase = fwid * _TPW
    lane = lax.iota(jnp.int32, 16)
    zero16 = jnp.zeros((16,), jnp.int32)
    ones16 = jnp.ones((16,), jnp.int32)

    fcps = [pltpu.async_copy(slct_t.at[e, pl.ds(fbase, _TPW)], fbuf.at[e], fsem)
            for e in range(_E)]

    for e in range(_E):
        pltpu.sync_copy(slct_t.at[e, pl.ds(base, _TPW)], sbuf.at[e])

    cntv[...] = zero16
    for g in range(_G):
        sl = pl.ds(g * 16, 16)
        best = sbuf[0, sl]
        bidx = zero16
        for e in range(1, _E):
            v = sbuf[e, sl]
            m = v > best
            bidx = jnp.where(m, _i16(e), bidx)
            best = jnp.where(m, v, best)
        prior = plsc.load_gather(cntv, [bidx])
        lr = zero16
        for e in range(_E):
            me = (bidx == _i16(e)).astype(jnp.int32)
            pfx = plsc.cumsum(me)
            lr = lr + me * (pfx - me)
        plsc.addupdate_scatter(cntv, [bidx], ones16)
        idxb[sl] = bidx
        rankb[sl] = prior + lr

    for cp in fcps:
        cp.wait()
    cntf[...] = zero16
    for g in range(_G):
        sl = pl.ds(g * 16, 16)
        best = fbuf[0, sl]
        bidx = zero16
        for e in range(1, _E):
            v = fbuf[e, sl]
            m = v > best
            bidx = jnp.where(m, _i16(e), bidx)
            best = jnp.where(m, v, best)
        plsc.addupdate_scatter(cntf, [bidx], ones16)

    pltpu.sync_copy(cntv, csh.at[pl.ds(wid * 16, 16)])
    pltpu.sync_copy(cntf, csh.at[pl.ds(fwid * 16, 16)])
    plsc.subcore_barrier()
    pltpu.sync_copy(csh, cntall)

    widv = jnp.full((16,), wid, jnp.int32)
    acc = zero16
    accp = zero16
    for w in range(_NW):
        ch = cntall[pl.ds(16 * w, 16)]
        acc = acc + ch
        accp = accp + jnp.where(_i16(w) < widv, ch, zero16)

    padded = (acc + _i16(_TILE - 1)) & _i16(~(_TILE - 1))
    ends = plsc.cumsum(padded)
    offs = ends - padded
    svm[...] = offs + accp
    endm[pl.ds(1, 16)] = ends
    fillm[pl.ds(1, 16)] = offs + acc

    for g in range(_G):
        sl = pl.ds(g * 16, 16)
        bidx = idxb[sl]
        slotb[sl] = rankb[sl] + plsc.load_gather(svm, [bidx])

    for k in range(_TPW):
        tmbuf[k, pl.ds(0, 16)] = jnp.full((16,), base + k, jnp.int32)
    pltpu.async_copy(tmbuf, tm_hbm.at[slotb], sem).wait()

    @pl.when(wid == 0)
    def _():
        ts0 = lane * _i16(_TILE)
        ts1 = (lane + _i16(16)) * _i16(_TILE)
        tev0 = zero16
        tev1 = zero16
        for e in range(_E - 1):
            endv = plsc.load_gather(endm, [_i16(e + 1)])
            tev0 = tev0 + (ts0 >= endv).astype(jnp.int32)
            tev1 = tev1 + (ts1 >= endv).astype(jnp.int32)
        fill0 = plsc.load_gather(fillm, [tev0 + ones16])
        fill1 = plsc.load_gather(fillm, [tev1 + ones16])
        vc0 = jnp.clip(fill0 - ts0, 0, _TILE)
        vc1 = jnp.clip(fill1 - ts1, 0, _TILE)
        teb[pl.ds(0, 16)] = tev0
        teb[pl.ds(16, 16)] = tev1
        teb[pl.ds(32, 16)] = vc0
        teb[pl.ds(48, 16)] = vc1
        pltpu.sync_copy(teb, meta_hbm)


def _route_scratch():
    return [
        pltpu.VMEM((_E, _TPW), jnp.float32),
        pltpu.VMEM((_E, _TPW), jnp.float32),
        pltpu.VMEM((_TPW,), jnp.int32),
        pltpu.VMEM((_TPW,), jnp.int32),
        pltpu.VMEM((_TPW,), jnp.int32),
        pltpu.VMEM((16,), jnp.int32),
        pltpu.VMEM((16,), jnp.int32),
        pltpu.VMEM((_NW * 16,), jnp.int32),
        pltpu.VMEM((16,), jnp.int32),
        pltpu.VMEM((32,), jnp.int32),
        pltpu.VMEM((32,), jnp.int32),
        pltpu.VMEM((64,), jnp.int32),
        pltpu.VMEM((_TPW, 128), jnp.int32),
        pltpu.VMEM_SHARED((_NW * 16,), jnp.int32),
        pltpu.SemaphoreType.DMA,
        pltpu.SemaphoreType.DMA,
    ]


_route = pl.kernel(
    _route_body,
    out_type=[
        jax.ShapeDtypeStruct((_C, 128), jnp.int32),
        jax.ShapeDtypeStruct((64,), jnp.int32),
    ],
    mesh=_MESH,
    compiler_params=pltpu.CompilerParams(needs_layout_passes=False),
    scratch_types=_route_scratch(),
)


def _mm_body(meta_ref, tm_ref, x_hbm, w_ref, b_ref, out_hbm, xg, yg,
             lsem, ssem):
    t = pl.program_id(0)
    vc = meta_ref[32 + t]

    @pl.when(vc > 0)
    def _():
        def issue_load(r, _):
            tok = tm_ref[t * _TILE + r]
            pltpu.make_async_copy(
                x_hbm.at[pl.ds(tok, 1)], xg.at[pl.ds(r, 1)], lsem).start()
            return _

        def drain_load(r, _):
            pltpu.make_async_copy(
                x_hbm.at[pl.ds(0, 1)], xg.at[pl.ds(0, 1)], lsem).wait()
            return _

        lax.fori_loop(0, vc, issue_load, 0)
        lax.fori_loop(0, vc, drain_load, 0)

        y = lax.dot_general(
            xg[...], w_ref[0],
            dimension_numbers=(((1,), (1,)), ((), ())),
            preferred_element_type=jnp.float32)
        yg[...] = jnp.maximum(y + b_ref[0], 0.0)

        def issue_store(r, _):
            tok = tm_ref[t * _TILE + r]
            pltpu.make_async_copy(
                yg.at[pl.ds(r, 1)], out_hbm.at[pl.ds(tok, 1)], ssem).start()
            return _

        def drain_store(r, _):
            pltpu.make_async_copy(
                yg.at[pl.ds(0, 1)], out_hbm.at[pl.ds(0, 1)], ssem).wait()
            return _

        lax.fori_loop(0, vc, issue_store, 0)
        lax.fori_loop(0, vc, drain_store, 0)


def _expert_matmul(meta, tok_map, x, W, b):
    grid_spec = pltpu.PrefetchScalarGridSpec(
        num_scalar_prefetch=2,
        grid=(_NT,),
        in_specs=[
            pl.BlockSpec(memory_space=pltpu.MemorySpace.HBM),
            pl.BlockSpec((1, _NHID, _NIN), lambda t, mt, tm: (mt[t], 0, 0)),
            pl.BlockSpec((1, 1, _NHID), lambda t, mt, tm: (mt[t], 0, 0)),
        ],
        out_specs=pl.BlockSpec(memory_space=pltpu.MemorySpace.HBM),
        scratch_shapes=[
            pltpu.VMEM((_TILE, _NIN), jnp.float32),
            pltpu.VMEM((_TILE, _NHID), jnp.float32),
            pltpu.SemaphoreType.DMA,
            pltpu.SemaphoreType.DMA,
        ],
    )
    return pl.pallas_call(
        _mm_body,
        grid_spec=grid_spec,
        out_shape=jax.ShapeDtypeStruct((_B, _NHID), jnp.float32),
    )(meta, tok_map, x, W, b.reshape(_E, 1, _NHID))


def kernel(x, slct, W, b):
    tm, meta = _route(slct.T)
    return _expert_matmul(meta, tm[:, 0], x, W, b)


# speedup vs baseline: 2.0473x; 1.6357x over previous
import jax
import jax.numpy as jnp
from jax import lax
from jax.experimental import pallas as pl
from jax.experimental.pallas import tpu as pltpu
from jax.experimental.pallas import tpu_sc as plsc

_B, _NIN, _NHID, _E = 2048, 1024, 1024, 8
_TILE = 128
_NT = 24
_C = _NT * _TILE

_NC, _NS = 2, 16
_NW = _NC * _NS
_TPW = _B // _NW
_G = _TPW // 16

_MESH = plsc.VectorSubcoreMesh(core_axis_name="c", subcore_axis_name="s",
                               num_cores=_NC, num_subcores=_NS)


def _i16(v):
    return jnp.full((16,), v, jnp.int32)


def _route_body(slct_t, x_hbm, xs_hbm, slot_hbm, te_hbm,
                sbuf, fbuf, idxb, rankb, slotb, cntv, cntf, cntall, svm, endm,
                teb, rows, csh, sem, rsem, fsem):
    cid = lax.axis_index("c")
    sid = lax.axis_index("s")
    wid = cid * _NS + sid
    fwid = (1 - cid) * _NS + sid
    base = wid * _TPW
    fbase = fwid * _TPW
    lane = lax.iota(jnp.int32, 16)
    zero16 = jnp.zeros((16,), jnp.int32)
    ones16 = jnp.ones((16,), jnp.int32)

    rowcp = pltpu.async_copy(x_hbm.at[pl.ds(base, _TPW)], rows, rsem)
    fcps = [pltpu.async_copy(slct_t.at[e, pl.ds(fbase, _TPW)], fbuf.at[e], fsem)
            for e in range(_E)]

    for e in range(_E):
        pltpu.sync_copy(slct_t.at[e, pl.ds(base, _TPW)], sbuf.at[e])

    cntv[...] = zero16
    for g in range(_G):
        sl = pl.ds(g * 16, 16)
        best = sbuf[0, sl]
        bidx = zero16
        for e in range(1, _E):
            v = sbuf[e, sl]
            m = v > best
            bidx = jnp.where(m, _i16(e), bidx)
            best = jnp.where(m, v, best)
        prior = plsc.load_gather(cntv, [bidx])
        lr = zero16
        for e in range(_E):
            me = (bidx == _i16(e)).astype(jnp.int32)
            pfx = plsc.cumsum(me)
            lr = lr + me * (pfx - me)
        plsc.addupdate_scatter(cntv, [bidx], ones16)
        idxb[sl] = bidx
        rankb[sl] = prior + lr

    for cp in fcps:
        cp.wait()
    cntf[...] = zero16
    for g in range(_G):
        sl = pl.ds(g * 16, 16)
        best = fbuf[0, sl]
        bidx = zero16
        for e in range(1, _E):
            v = fbuf[e, sl]
            m = v > best
            bidx = jnp.where(m, _i16(e), bidx)
            best = jnp.where(m, v, best)
        plsc.addupdate_scatter(cntf, [bidx], ones16)

    pltpu.sync_copy(cntv, csh.at[pl.ds(wid * 16, 16)])
    pltpu.sync_copy(cntf, csh.at[pl.ds(fwid * 16, 16)])
    plsc.subcore_barrier()
    pltpu.sync_copy(csh, cntall)

    widv = jnp.full((16,), wid, jnp.int32)
    acc = zero16
    accp = zero16
    for w in range(_NW):
        ch = cntall[pl.ds(16 * w, 16)]
        acc = acc + ch
        accp = accp + jnp.where(_i16(w) < widv, ch, zero16)

    padded = (acc + _i16(_TILE - 1)) & _i16(~(_TILE - 1))
    ends = plsc.cumsum(padded)
    svm[...] = ends - padded + accp
    endm[pl.ds(1, 16)] = ends

    for g in range(_G):
        sl = pl.ds(g * 16, 16)
        bidx = idxb[sl]
        slotb[sl] = rankb[sl] + plsc.load_gather(svm, [bidx])
    pltpu.sync_copy(slotb, slot_hbm.at[pl.ds(base, _TPW)])

    rowcp.wait()
    pltpu.async_copy(rows, xs_hbm.at[slotb], sem).wait()

    @pl.when(wid == 0)
    def _():
        ts0 = lane * _i16(_TILE)
        ts1 = (lane + _i16(16)) * _i16(_TILE)
        tev0 = zero16
        tev1 = zero16
        for e in range(_E - 1):
            endv = plsc.load_gather(endm, [_i16(e + 1)])
            tev0 = tev0 + (ts0 >= endv).astype(jnp.int32)
            tev1 = tev1 + (ts1 >= endv).astype(jnp.int32)
        used = plsc.load_gather(endm, [_i16(_E)])
        teb[pl.ds(0, 16)] = tev0
        teb[pl.ds(16, 16)] = tev1
        teb[pl.ds(32, 16)] = (lane == zero16).astype(jnp.int32) * used
        pltpu.sync_copy(teb, te_hbm)


def _route_scratch():
    return [
        pltpu.VMEM((_E, _TPW), jnp.float32),
        pltpu.VMEM((_E, _TPW), jnp.float32),
        pltpu.VMEM((_TPW,), jnp.int32),
        pltpu.VMEM((_TPW,), jnp.int32),
        pltpu.VMEM((_TPW,), jnp.int32),
        pltpu.VMEM((16,), jnp.int32),
        pltpu.VMEM((16,), jnp.int32),
        pltpu.VMEM((_NW * 16,), jnp.int32),
        pltpu.VMEM((16,), jnp.int32),
        pltpu.VMEM((32,), jnp.int32),
        pltpu.VMEM((48,), jnp.int32),
        pltpu.VMEM((_TPW, _NIN), jnp.float32),
        pltpu.VMEM_SHARED((_NW * 16,), jnp.int32),
        pltpu.SemaphoreType.DMA,
        pltpu.SemaphoreType.DMA,
        pltpu.SemaphoreType.DMA,
    ]


_route = pl.kernel(
    _route_body,
    out_type=[
        jax.ShapeDtypeStruct((_C, _NIN), jnp.float32),
        jax.ShapeDtypeStruct((_B,), jnp.int32),
        jax.ShapeDtypeStruct((48,), jnp.int32),
    ],
    mesh=_MESH,
    compiler_params=pltpu.CompilerParams(needs_layout_passes=False),
    scratch_types=_route_scratch(),
)


def _gather_body(slot_hbm, ys_hbm, out_hbm, slotv, rows, sem):
    wid = lax.axis_index("s") * _NC + lax.axis_index("c")
    base = wid * _TPW
    pltpu.sync_copy(slot_hbm.at[pl.ds(base, _TPW)], slotv)
    pltpu.async_copy(ys_hbm.at[slotv], rows, sem).wait()
    pltpu.sync_copy(rows, out_hbm.at[pl.ds(base, _TPW)])


_gather = pl.kernel(
    _gather_body,
    out_type=jax.ShapeDtypeStruct((_B, _NHID), jnp.float32),
    mesh=_MESH,
    compiler_params=pltpu.CompilerParams(needs_layout_passes=False),
    scratch_types=[
        pltpu.VMEM((_TPW,), jnp.int32),
        pltpu.VMEM((_TPW, _NHID), jnp.float32),
        pltpu.SemaphoreType.DMA,
    ],
)


def _mm_body(te_ref, used_ref, x_ref, w_ref, b_ref, o_ref):
    @pl.when(pl.program_id(0) * _TILE < used_ref[0])
    def _():
        y = lax.dot_general(
            x_ref[...], w_ref[0],
            dimension_numbers=(((1,), (1,)), ((), ())),
            preferred_element_type=jnp.float32)
        o_ref[...] = jnp.maximum(y + b_ref[0], 0.0)


def _expert_matmul(x_sorted, tile_expert, used, W, b):
    def _cap(t, u):
        return jnp.minimum(t, (u[0] + _TILE - 1) // _TILE - 1)

    grid_spec = pltpu.PrefetchScalarGridSpec(
        num_scalar_prefetch=2,
        grid=(_NT,),
        in_specs=[
            pl.BlockSpec((_TILE, _NIN), lambda t, te, u: (_cap(t, u), 0)),
            pl.BlockSpec((1, _NHID, _NIN), lambda t, te, u: (te[t], 0, 0)),
            pl.BlockSpec((1, 1, _NHID), lambda t, te, u: (te[t], 0, 0)),
        ],
        out_specs=pl.BlockSpec((_TILE, _NHID), lambda t, te, u: (_cap(t, u), 0)),
    )
    return pl.pallas_call(
        _mm_body,
        grid_spec=grid_spec,
        out_shape=jax.ShapeDtypeStruct((_C, _NHID), jnp.float32),
    )(tile_expert, used, x_sorted, W, b.reshape(_E, 1, _NHID))


def kernel(x, slct, W, b):
    x_sorted, slot, te = _route(slct.T, x)
    y_sorted = _expert_matmul(x_sorted, te[:32], te[32:40], W, b)
    return _gather(slot, y_sorted)
